# Initial kernel scaffold; baseline (speedup 1.0000x reference)
import jax
import jax.numpy as jnp
from jax.experimental import pallas as pl


def kernel(x, edge_index, W1, b1, W2, b2):
    raise NotImplementedError("write your pallas kernel here")


# speedup vs baseline: 4.0188x; 4.0188x over previous
import functools

import jax
import jax.numpy as jnp
from jax import lax
from jax.experimental import pallas as pl
from jax.experimental.pallas import tpu as pltpu
from jax.experimental.pallas import tpu_sc as plsc

N_NODES = 10000
D = 128
N_EDGES = 320000

NC = 2
NS = 16
NW = NC * NS

CHUNK = 128
CPT = (N_EDGES + NW * CHUNK - 1) // (NW * CHUNK)
E_PAD = NW * CPT * CHUNK

ACC_ROWS = 10240
ZROWS = ACC_ROWS // NS

ROW_BLOCK = 400
TC_GRID = N_NODES // ROW_BLOCK


def _spmm_partials(h, src3, dst3, zeros_hbm):
    mesh = plsc.VectorSubcoreMesh(core_axis_name="c", subcore_axis_name="s")

    @functools.partial(
        pl.kernel,
        mesh=mesh,
        out_type=jax.ShapeDtypeStruct((NC, ACC_ROWS, D), jnp.float32),
        scratch_types=[
            pltpu.VMEM((CPT, CHUNK), jnp.int32),
            pltpu.VMEM((CPT, CHUNK), jnp.int32),
            pltpu.VMEM((CHUNK, D), jnp.float32),
            pltpu.VMEM_SHARED((ACC_ROWS, D), jnp.float32),
            pltpu.SemaphoreType.DMA,
        ],
    )
    def k(h_hbm, src_hbm, dst_hbm, z_hbm, out_hbm, src_v, dst_v, rows_v, acc_sh, sem):
        cid = lax.axis_index("c")
        sid = lax.axis_index("s")
        wid = sid * NC + cid

        pltpu.sync_copy(z_hbm, acc_sh.at[pl.ds(sid * ZROWS, ZROWS)])
        pltpu.sync_copy(src_hbm.at[wid], src_v)
        pltpu.sync_copy(dst_hbm.at[wid], dst_v)
        plsc.subcore_barrier()

        def body(j, carry):
            pltpu.async_copy(h_hbm.at[src_v.at[j]], rows_v, sem).wait()
            pltpu.sync_copy(rows_v, acc_sh.at[dst_v.at[j]], add=True)
            return carry

        lax.fori_loop(0, CPT, body, 0)

        plsc.subcore_barrier()
        pltpu.sync_copy(
            acc_sh.at[pl.ds(sid * ZROWS, ZROWS)],
            out_hbm.at[cid].at[pl.ds(sid * ZROWS, ZROWS)],
        )

    return k(h, src3, dst3, zeros_hbm)


def _tc_layer1(p0, p1, W1, b1):
    def body(p0_ref, p1_ref, w_ref, b_ref, o_ref):
        s = p0_ref[...] + p1_ref[...]
        t = jnp.dot(s, w_ref[...], preferred_element_type=jnp.float32)
        o_ref[...] = jnp.maximum(t + b_ref[...], 0.0)

    return pl.pallas_call(
        body,
        grid=(TC_GRID,),
        in_specs=[
            pl.BlockSpec((ROW_BLOCK, D), lambda i: (i, 0)),
            pl.BlockSpec((ROW_BLOCK, D), lambda i: (i, 0)),
            pl.BlockSpec((D, D), lambda i: (0, 0)),
            pl.BlockSpec((1, D), lambda i: (0, 0)),
        ],
        out_specs=pl.BlockSpec((ROW_BLOCK, D), lambda i: (i, 0)),
        out_shape=jax.ShapeDtypeStruct((N_NODES, D), jnp.float32),
    )(p0, p1, W1, b1.reshape(1, D))


def _tc_layer2(q0, q1, W2, b2):
    def body(q0_ref, q1_ref, w_ref, b_ref, o_ref):
        s = q0_ref[...] + q1_ref[...]
        z = jnp.dot(s, w_ref[...], preferred_element_type=jnp.float32)
        z = z + b_ref[...]
        m = jnp.max(z, axis=1, keepdims=True)
        e = z - m
        lse = jnp.log(jnp.sum(jnp.exp(e), axis=1, keepdims=True))
        o_ref[...] = e - lse

    return pl.pallas_call(
        body,
        grid=(TC_GRID,),
        in_specs=[
            pl.BlockSpec((ROW_BLOCK, D), lambda i: (i, 0)),
            pl.BlockSpec((ROW_BLOCK, D), lambda i: (i, 0)),
            pl.BlockSpec((D, D), lambda i: (0, 0)),
            pl.BlockSpec((1, D), lambda i: (0, 0)),
        ],
        out_specs=pl.BlockSpec((ROW_BLOCK, D), lambda i: (i, 0)),
        out_shape=jax.ShapeDtypeStruct((N_NODES, D), jnp.float32),
    )(q0, q1, W2, b2.reshape(1, D))


def kernel(x, edge_index, W1, b1, W2, b2):
    src = edge_index[0].astype(jnp.int32)
    dst = edge_index[1].astype(jnp.int32)
    pad = E_PAD - N_EDGES
    src = jnp.concatenate([src, jnp.zeros((pad,), jnp.int32)])
    dst = jnp.concatenate([dst, jnp.full((pad,), N_NODES, jnp.int32)])
    src3 = src.reshape(NW, CPT, CHUNK)
    dst3 = dst.reshape(NW, CPT, CHUNK)
    zeros_hbm = jnp.zeros((ZROWS, D), jnp.float32)

    P = _spmm_partials(x, src3, dst3, zeros_hbm)
    h = _tc_layer1(P[0], P[1], W1, b1)
    Q = _spmm_partials(h, src3, dst3, zeros_hbm)
    return _tc_layer2(Q[0], Q[1], W2, b2)


# speedup vs baseline: 5.1777x; 1.2884x over previous
import functools

import jax
import jax.numpy as jnp
from jax import lax
from jax.experimental import pallas as pl
from jax.experimental.pallas import tpu as pltpu
from jax.experimental.pallas import tpu_sc as plsc

N_NODES = 10000
D = 128
N_EDGES = 320000

NC = 2
NS = 16
NW = NC * NS

CHUNK = 112
CPT = 90
HALF1 = 48
HALF2 = CPT - HALF1
E_PAD = NW * CPT * CHUNK

ACC_ROWS = 10112
ZROWS = ACC_ROWS // NS

ROW_BLOCK = 400
TC_GRID = N_NODES // ROW_BLOCK


def _spmm_partials(h, src3, dst3, zeros_hbm):
    mesh = plsc.VectorSubcoreMesh(core_axis_name="c", subcore_axis_name="s")

    @functools.partial(
        pl.kernel,
        mesh=mesh,
        out_type=jax.ShapeDtypeStruct((NC, ACC_ROWS, D), jnp.float32),
        scratch_types=[
            pltpu.VMEM((HALF1, CHUNK), jnp.int32),
            pltpu.VMEM((HALF1, CHUNK), jnp.int32),
            pltpu.VMEM((2, CHUNK, D), jnp.float32),
            pltpu.VMEM_SHARED((ACC_ROWS, D), jnp.float32),
            pltpu.SemaphoreType.DMA,
            pltpu.SemaphoreType.DMA,
        ],
    )
    def k(h_hbm, src_hbm, dst_hbm, z_hbm, out_hbm, src_v, dst_v, rows_v, acc_sh,
          gsem, ssem):
        cid = lax.axis_index("c")
        sid = lax.axis_index("s")
        wid = sid * NC + cid

        pltpu.sync_copy(z_hbm, acc_sh.at[pl.ds(sid * ZROWS, ZROWS)])
        plsc.subcore_barrier()

        def start_g(j, b):
            pltpu.async_copy(h_hbm.at[src_v.at[j]], rows_v.at[b], gsem)

        def wait_g(j, b):
            pltpu.make_async_copy(h_hbm.at[src_v.at[j]], rows_v.at[b], gsem).wait()

        def start_s(j, b):
            pltpu.async_copy(rows_v.at[b], acc_sh.at[dst_v.at[j]], ssem, add=True)

        def wait_s(j, b):
            pltpu.make_async_copy(rows_v.at[b], acc_sh.at[dst_v.at[j]], ssem).wait()

        def phase(off, n):
            pltpu.sync_copy(src_hbm.at[wid].at[pl.ds(off, n)],
                            src_v.at[pl.ds(0, n)])
            pltpu.sync_copy(dst_hbm.at[wid].at[pl.ds(off, n)],
                            dst_v.at[pl.ds(0, n)])

            start_g(0, 0)
            wait_g(0, 0)
            start_s(0, 0)
            start_g(1, 1)

            def body(j, carry):
                b = j % 2
                wait_g(j, b)
                start_s(j, b)
                wait_s(j - 1, 1 - b)
                start_g(j + 1, 1 - b)
                return carry

            lax.fori_loop(1, n - 1, body, 0)

            j = n - 1
            b = j % 2
            wait_g(j, b)
            start_s(j, b)
            wait_s(j - 1, 1 - b)
            wait_s(j, b)

        phase(0, HALF1)
        phase(HALF1, HALF2)

        plsc.subcore_barrier()
        pltpu.sync_copy(
            acc_sh.at[pl.ds(sid * ZROWS, ZROWS)],
            out_hbm.at[cid].at[pl.ds(sid * ZROWS, ZROWS)],
        )

    return k(h, src3, dst3, zeros_hbm)


def _tc_layer1(p0, p1, W1, b1):
    def body(p0_ref, p1_ref, w_ref, b_ref, o_ref):
        s = p0_ref[...] + p1_ref[...]
        t = jnp.dot(s, w_ref[...], preferred_element_type=jnp.float32)
        o_ref[...] = jnp.maximum(t + b_ref[...], 0.0)

    return pl.pallas_call(
        body,
        grid=(TC_GRID,),
        in_specs=[
            pl.BlockSpec((ROW_BLOCK, D), lambda i: (i, 0)),
            pl.BlockSpec((ROW_BLOCK, D), lambda i: (i, 0)),
            pl.BlockSpec((D, D), lambda i: (0, 0)),
            pl.BlockSpec((1, D), lambda i: (0, 0)),
        ],
        out_specs=pl.BlockSpec((ROW_BLOCK, D), lambda i: (i, 0)),
        out_shape=jax.ShapeDtypeStruct((N_NODES, D), jnp.float32),
    )(p0, p1, W1, b1.reshape(1, D))


def _tc_layer2(q0, q1, W2, b2):
    def body(q0_ref, q1_ref, w_ref, b_ref, o_ref):
        s = q0_ref[...] + q1_ref[...]
        z = jnp.dot(s, w_ref[...], preferred_element_type=jnp.float32)
        z = z + b_ref[...]
        m = jnp.max(z, axis=1, keepdims=True)
        e = z - m
        lse = jnp.log(jnp.sum(jnp.exp(e), axis=1, keepdims=True))
        o_ref[...] = e - lse

    return pl.pallas_call(
        body,
        grid=(TC_GRID,),
        in_specs=[
            pl.BlockSpec((ROW_BLOCK, D), lambda i: (i, 0)),
            pl.BlockSpec((ROW_BLOCK, D), lambda i: (i, 0)),
            pl.BlockSpec((D, D), lambda i: (0, 0)),
            pl.BlockSpec((1, D), lambda i: (0, 0)),
        ],
        out_specs=pl.BlockSpec((ROW_BLOCK, D), lambda i: (i, 0)),
        out_shape=jax.ShapeDtypeStruct((N_NODES, D), jnp.float32),
    )(q0, q1, W2, b2.reshape(1, D))


def kernel(x, edge_index, W1, b1, W2, b2):
    src = edge_index[0].astype(jnp.int32)
    dst = edge_index[1].astype(jnp.int32)
    pad = E_PAD - N_EDGES
    src = jnp.concatenate([src, jnp.zeros((pad,), jnp.int32)])
    dst = jnp.concatenate([dst, jnp.full((pad,), N_NODES, jnp.int32)])
    src3 = src.reshape(NW, CPT, CHUNK)
    dst3 = dst.reshape(NW, CPT, CHUNK)
    zeros_hbm = jnp.zeros((ZROWS, D), jnp.float32)

    P = _spmm_partials(x, src3, dst3, zeros_hbm)
    h = _tc_layer1(P[0], P[1], W1, b1)
    Q = _spmm_partials(h, src3, dst3, zeros_hbm)
    return _tc_layer2(Q[0], Q[1], W2, b2)
